# Initial kernel scaffold; baseline (speedup 1.0000x reference)
import jax
import jax.numpy as jnp
from jax.experimental import pallas as pl


def kernel(x, edge_weights):
    raise NotImplementedError("write your pallas kernel here")


# speedup vs baseline: 1.3506x; 1.3506x over previous
import functools

import jax
import jax.numpy as jnp
from jax.experimental import pallas as pl

_BLK = 256


def _co_occurrence_block(ew_ref, x_ref, xi_ref, out_ref):
    out_ref[...] = xi_ref[...]
    nz = jnp.any(ew_ref[...] != 0.0)

    @pl.when(nz)
    def _():
        ew = ew_ref[...]
        for b in range(x_ref.shape[0]):
            out_ref[b, :, :] += jnp.dot(
                ew, x_ref[b, :, :], preferred_element_type=jnp.float32
            )


@functools.partial(jax.jit, static_argnames=())
def kernel(x, edge_weights):
    B, C, F = x.shape
    grid = (C // _BLK,)
    return pl.pallas_call(
        _co_occurrence_block,
        grid=grid,
        in_specs=[
            pl.BlockSpec((_BLK, C), lambda i: (i, 0)),
            pl.BlockSpec((B, C, F), lambda i: (0, 0, 0)),
            pl.BlockSpec((B, _BLK, F), lambda i: (0, i, 0)),
        ],
        out_specs=pl.BlockSpec((B, _BLK, F), lambda i: (0, i, 0)),
        out_shape=jax.ShapeDtypeStruct((B, C, F), x.dtype),
    )(edge_weights, x, x)
